# baseline (speedup 1.0000x reference)
import functools

import jax
import jax.numpy as jnp
from jax import lax
from jax.experimental import pallas as pl
from jax.experimental.pallas import tpu as pltpu
from jax.experimental.pallas import tpu_sc as plsc

NUM_EXPERTS = 8
TOP_K = 2
BS = 256
SC_CORES = 2
SC_SUBCORES = 16
NW = SC_CORES * SC_SUBCORES


def _router_kernel(x_ref, gw_ref, ei_ref, ew_ref):
    x = x_ref[...]
    gw = gw_ref[...]
    logits = jax.lax.dot_general(
        x, gw, (((1,), (1,)), ((), ())),
        preferred_element_type=jnp.float32,
        precision=jax.lax.Precision.DEFAULT)
    p = jax.nn.softmax(logits, axis=-1)
    e_dim = p.shape[-1]
    iota = jax.lax.broadcasted_iota(jnp.int32, p.shape, 1)
    m1 = jnp.max(p, axis=-1, keepdims=True)
    i1 = jnp.min(jnp.where(p == m1, iota, e_dim), axis=-1, keepdims=True)
    mask1 = iota == i1
    pm = jnp.where(mask1, -jnp.inf, p)
    m2 = jnp.max(pm, axis=-1, keepdims=True)
    i2 = jnp.min(jnp.where(pm == m2, iota, e_dim), axis=-1, keepdims=True)
    denom = m1 + m2
    ei_ref[...] = jnp.concatenate([i1, i2], axis=1)
    ew_ref[...] = jnp.concatenate([m1 / denom, m2 / denom], axis=1)


def _gmm_kernel(be_ref, xs_ref, wrow_ref, gp_ref, up_ref, dp_ref, ys_ref):
    del be_ref
    xb = xs_ref[...]
    g = jax.lax.dot_general(
        xb, gp_ref[0], (((1,), (1,)), ((), ())),
        preferred_element_type=jnp.float32,
        precision=jax.lax.Precision.DEFAULT)
    u = jax.lax.dot_general(
        xb, up_ref[0], (((1,), (1,)), ((), ())),
        preferred_element_type=jnp.float32,
        precision=jax.lax.Precision.DEFAULT)
    h = (g * jax.lax.logistic(g)) * u
    y = jax.lax.dot_general(
        h, dp_ref[0], (((1,), (1,)), ((), ())),
        preferred_element_type=jnp.float32,
        precision=jax.lax.Precision.DEFAULT)
    ys_ref[...] = y * wrow_ref[...]


def _sc_gather(x, gidx, p_max, d):
    b_per_w = p_max // NW
    cr = 64
    mesh = plsc.VectorSubcoreMesh(core_axis_name="c", subcore_axis_name="s")

    @functools.partial(
        pl.kernel, mesh=mesh,
        out_type=jax.ShapeDtypeStruct((p_max, d), jnp.float32),
        scratch_types=[
            pltpu.VMEM((b_per_w,), jnp.int32),
            pltpu.VMEM((cr, d), jnp.float32),
            pltpu.SemaphoreType.DMA,
        ],
    )
    def k(x_hbm, idx_hbm, out_hbm, idx_v, rows_v, sem):
        wid = lax.axis_index("s") * SC_CORES + lax.axis_index("c")
        base = wid * b_per_w
        pltpu.sync_copy(idx_hbm.at[pl.ds(base, b_per_w)], idx_v)

        @pl.loop(0, b_per_w, step=cr)
        def _(c):
            pltpu.async_copy(
                x_hbm.at[idx_v.at[pl.ds(c, cr)]], rows_v, sem).wait()
            pltpu.sync_copy(rows_v, out_hbm.at[pl.ds(base + c, cr)])

    return k(x, gidx)


def _sc_combine(ys, pos0, pos1, t, d):
    t_per_w = t // NW
    ct = 32
    mesh = plsc.VectorSubcoreMesh(core_axis_name="c", subcore_axis_name="s")

    @functools.partial(
        pl.kernel, mesh=mesh,
        out_type=jax.ShapeDtypeStruct((t, d), jnp.float32),
        scratch_types=[
            pltpu.VMEM((t_per_w,), jnp.int32),
            pltpu.VMEM((t_per_w,), jnp.int32),
            pltpu.VMEM((ct, d), jnp.float32),
            pltpu.VMEM((ct, d), jnp.float32),
            pltpu.SemaphoreType.DMA,
            pltpu.SemaphoreType.DMA,
        ],
    )
    def k(ys_hbm, p0_hbm, p1_hbm, out_hbm, p0_v, p1_v, r0_v, r1_v, s0, s1):
        wid = lax.axis_index("s") * SC_CORES + lax.axis_index("c")
        base = wid * t_per_w
        pltpu.sync_copy(p0_hbm.at[pl.ds(base, t_per_w)], p0_v)
        pltpu.sync_copy(p1_hbm.at[pl.ds(base, t_per_w)], p1_v)

        @pl.loop(0, t_per_w, step=ct)
        def _(c):
            cp0 = pltpu.async_copy(
                ys_hbm.at[p0_v.at[pl.ds(c, ct)]], r0_v, s0)
            cp1 = pltpu.async_copy(
                ys_hbm.at[p1_v.at[pl.ds(c, ct)]], r1_v, s1)
            cp0.wait()
            cp1.wait()

            @pl.loop(0, ct)
            def _(r):
                @pl.loop(0, d, step=16)
                def _(j):
                    r0_v.at[pl.ds(r, 1), pl.ds(j, 16)][...] = (
                        r0_v.at[pl.ds(r, 1), pl.ds(j, 16)][...]
                        + r1_v.at[pl.ds(r, 1), pl.ds(j, 16)][...])

            pltpu.sync_copy(r0_v, out_hbm.at[pl.ds(base + c, ct)])

    return k(ys, pos0, pos1)


@functools.partial(jax.jit, static_argnames=())
def kernel(hidden_states, gate_w, gate_proj, up_proj, down_proj):
    b, s, h = hidden_states.shape
    x = hidden_states.reshape(-1, h)
    t = x.shape[0]
    f = gate_proj.shape[1]
    n_pairs = t * TOP_K
    p_max = n_pairs + NUM_EXPERTS * BS
    nb = p_max // BS

    ei, ew = pl.pallas_call(
        _router_kernel,
        out_shape=(jax.ShapeDtypeStruct((t, TOP_K), jnp.int32),
                   jax.ShapeDtypeStruct((t, TOP_K), jnp.float32)),
    )(x, gate_w)

    eflat = ei.reshape(-1)
    wflat = ew.reshape(-1)
    onehot = (eflat[:, None] == jnp.arange(NUM_EXPERTS)[None, :]
              ).astype(jnp.int32)
    ranks = jnp.cumsum(onehot, axis=0) - 1
    myrank = jnp.take_along_axis(ranks, eflat[:, None], axis=1)[:, 0]
    counts = onehot.sum(axis=0)
    padded = ((counts + BS - 1) // BS) * BS
    cum_pad = jnp.cumsum(padded)
    pad_off = cum_pad - padded
    dst = pad_off[eflat] + myrank
    gidx = jnp.zeros((p_max,), jnp.int32).at[dst].set(
        jnp.arange(n_pairs, dtype=jnp.int32) // TOP_K)
    wrow = jnp.zeros((p_max,), jnp.float32).at[dst].set(wflat)
    blk_start = jnp.arange(nb, dtype=jnp.int32) * BS
    be = jnp.minimum(
        jnp.searchsorted(cum_pad, blk_start, side="right"),
        NUM_EXPERTS - 1).astype(jnp.int32)
    pos = dst.reshape(t, TOP_K)
    pos0 = pos[:, 0]
    pos1 = pos[:, 1]

    xs = _sc_gather(x, gidx, p_max, h)

    grid_spec = pltpu.PrefetchScalarGridSpec(
        num_scalar_prefetch=1,
        grid=(nb,),
        in_specs=[
            pl.BlockSpec((BS, h), lambda i, be_r: (i, 0)),
            pl.BlockSpec((BS, 1), lambda i, be_r: (i, 0)),
            pl.BlockSpec((1, f, h), lambda i, be_r: (be_r[i], 0, 0)),
            pl.BlockSpec((1, f, h), lambda i, be_r: (be_r[i], 0, 0)),
            pl.BlockSpec((1, h, f), lambda i, be_r: (be_r[i], 0, 0)),
        ],
        out_specs=pl.BlockSpec((BS, h), lambda i, be_r: (i, 0)),
    )
    ys = pl.pallas_call(
        _gmm_kernel,
        grid_spec=grid_spec,
        out_shape=jax.ShapeDtypeStruct((p_max, h), jnp.float32),
    )(be, xs, wrow[:, None], gate_proj, up_proj, down_proj)

    out = _sc_combine(ys, pos0, pos1, t, h)
    return out.reshape(b, s, h)
